# Initial kernel scaffold; baseline (speedup 1.0000x reference)
import jax
import jax.numpy as jnp
from jax.experimental import pallas as pl


def kernel(observations, params):
    raise NotImplementedError("write your pallas kernel here")


# speedup vs baseline: 1.6293x; 1.6293x over previous
import jax
import jax.numpy as jnp
from jax.experimental import pallas as pl
from jax.experimental.pallas import tpu as pltpu

N_NODES = 48
CH = 32
B = 64


def _bn(x, g, b):
    m = jnp.mean(x, axis=0, keepdims=True)
    v = jnp.mean((x - m) ** 2, axis=0, keepdims=True)
    return (x - m) / jnp.sqrt(v + 1e-5) * g + b


def _ln_relu(x, g, b):
    m = jnp.mean(x, axis=-1, keepdims=True)
    v = jnp.mean((x - m) ** 2, axis=-1, keepdims=True)
    return jax.nn.relu((x - m) / jnp.sqrt(v + 1e-5) * g + b)


def _dot_bf16(a, b):
    return jnp.dot(a.astype(jnp.bfloat16), b.astype(jnp.bfloat16),
                   preferred_element_type=jnp.float32)


def _conv_mlp(t, Ws, bs, gs, betas):
    for li in range(4):
        t = _dot_bf16(t, Ws[li]) + bs[li]
        if li < 3:
            t = jax.nn.relu(_bn(t, gs[li], betas[li]))
    return t


def _lift0_kernel(obs_ref, eW_ref, eb_ref, out_ref):
    out_ref[...] = jax.nn.relu(1.0 + obs_ref[...] * eW_ref[0, 0] + eb_ref[0, 0])


def _conv0_mlp_xla(h0, c):
    t = h0 @ c["Ws"][0] + c["bs"][0]
    for li in range(4):
        if li > 0:
            t = t @ c["Ws"][li] + c["bs"][li]
        if li < 3:
            m = jnp.mean(t, axis=0, keepdims=True)
            v = jnp.var(t, axis=0, keepdims=True)
            t = (t - m) / jnp.sqrt(v + 1e-5) * c["gammas"][li] + c["betas"][li]
            t = jax.nn.relu(t)
    return t


def _unpack(prefs):
    eW, eb = prefs[0][...], prefs[1][...]
    Ws = [r[...] for r in prefs[2:6]]
    bs = [r[...] for r in prefs[6:10]]
    gs = [r[...] for r in prefs[10:13]]
    betas = [r[...] for r in prefs[13:16]]
    return (eW, eb), Ws, bs, gs, betas


def _convN_kernel(nhalf, jk_in, obsT_ref, x_ref, x2d_ref, *refs):
    out_ref = refs[-1]
    jk_refs = refs[16:-1] if jk_in else ()
    (eW, eb), Ws, bs, gs, betas = _unpack(refs[:16])
    obsT = obsT_ref[...]
    x = x_ref[...]
    x2d = x2d_ref[...]
    M = B * N_NODES
    W = N_NODES * CH // nhalf

    aggr = jnp.zeros((M, CH), jnp.float32)
    for h in range(nhalf):
        qg = jax.lax.broadcasted_iota(jnp.int32, (N_NODES, W), 1) + h * W
        jr = jax.lax.broadcasted_iota(jnp.int32, (N_NODES, W), 0)
        csel = jax.lax.broadcasted_iota(jnp.int32, (CH, W), 1) + h * W
        crow = jax.lax.broadcasted_iota(jnp.int32, (CH, W), 0)
        Tsel = (csel % CH == crow).astype(jnp.float32)
        wtile = jnp.dot(eW, Tsel, preferred_element_type=jnp.float32, precision=jax.lax.Precision.HIGHEST)
        ebtile = jnp.dot(eb, Tsel, preferred_element_type=jnp.float32, precision=jax.lax.Precision.HIGHEST)
        R = jnp.where(jr == qg // CH, wtile, 0.0)
        O = jnp.dot(obsT, R, preferred_element_type=jnp.float32, precision=jax.lax.Precision.HIGHEST)
        xb = jnp.broadcast_to(
            x2d[:, None, h * W:(h + 1) * W], (B, N_NODES, W)).reshape(M, W)
        sq = jax.lax.broadcasted_iota(jnp.int32, (W, CH), 0) + h * W
        sc = jax.lax.broadcasted_iota(jnp.int32, (W, CH), 1)
        S = (sq % CH == sc).astype(jnp.float32)
        Z = jax.nn.relu(xb + ebtile + O)
        aggr = aggr + jnp.dot(Z, S, preferred_element_type=jnp.float32, precision=jax.lax.Precision.HIGHEST)
    hcat = x + aggr
    t = _conv_mlp(hcat, Ws, bs, gs, betas)
    for jr_ref in jk_refs:
        t = jnp.maximum(t, jr_ref[...])
    out_ref[...] = t


def _conv_args(c):
    args = [c["eW"], c["eb"].reshape(1, -1)]
    args.extend(c["Ws"])
    args.extend(b.reshape(1, -1) for b in c["bs"])
    args.extend(g.reshape(1, -1) for g in c["gammas"])
    args.extend(b.reshape(1, -1) for b in c["betas"])
    return args


def _run_convs(observations, obsT, convs):
    import functools
    M = B * N_NODES
    oshape = jax.ShapeDtypeStruct((M, CH), jnp.float32)
    c0 = convs[0]
    msgE = pl.pallas_call(
        _lift0_kernel,
        out_shape=jax.ShapeDtypeStruct(observations.shape, jnp.float32),
    )(observations, c0["eW"], c0["eb"].reshape(1, -1))
    dst = jnp.arange(observations.shape[1]) % N_NODES
    aggr0 = jnp.zeros((B, N_NODES), jnp.float32).at[:, dst].add(msgE)
    h0 = (1.0 + aggr0).reshape(M, 1)
    x1 = _conv0_mlp_xla(h0, convs[0])
    x2 = pl.pallas_call(functools.partial(_convN_kernel, 2, False),
                        out_shape=oshape)(
        obsT, x1, x1.reshape(B, N_NODES * CH), *_conv_args(convs[1]))
    jk = pl.pallas_call(functools.partial(_convN_kernel, 2, True),
                        out_shape=oshape)(
        obsT, x2, x2.reshape(B, N_NODES * CH), *_conv_args(convs[2]), x1, x2)
    return jk


def _gemm_bias_kernel(x_ref, w_ref, b_ref, o_ref):
    o_ref[...] = (jnp.dot(x_ref[...], w_ref[...],
                          preferred_element_type=jnp.float32, precision=jax.lax.Precision.HIGHEST) + b_ref[...])


def _gemm_bias(x, w, b, nt):
    K, N = w.shape
    return pl.pallas_call(
        _gemm_bias_kernel,
        grid=(N // nt,),
        in_specs=[
            pl.BlockSpec((x.shape[0], K), lambda i: (0, 0)),
            pl.BlockSpec((K, nt), lambda i: (0, i)),
            pl.BlockSpec((1, nt), lambda i: (0, i)),
        ],
        out_specs=pl.BlockSpec((x.shape[0], nt), lambda i: (0, i)),
        out_shape=jax.ShapeDtypeStruct((x.shape[0], N), jnp.float32),
    )(x, w, b.reshape(1, -1))


def _ln_gemm_kernel(x_ref, g_ref, be_ref, w_ref, b_ref, o_ref, a_scr):
    @pl.when(pl.program_id(0) == 0)
    def _():
        a_scr[...] = _ln_relu(x_ref[...], g_ref[...], be_ref[...])

    o_ref[...] = (jnp.dot(a_scr[...], w_ref[...],
                          preferred_element_type=jnp.float32, precision=jax.lax.Precision.HIGHEST) + b_ref[...])


def _ln_gemm_bias(x, g, be, w, b, nt):
    M, K = x.shape
    N = w.shape[1]
    return pl.pallas_call(
        _ln_gemm_kernel,
        grid=(N // nt,),
        in_specs=[
            pl.BlockSpec((M, K), lambda i: (0, 0)),
            pl.BlockSpec((1, K), lambda i: (0, 0)),
            pl.BlockSpec((1, K), lambda i: (0, 0)),
            pl.BlockSpec((K, nt), lambda i: (0, i)),
            pl.BlockSpec((1, nt), lambda i: (0, i)),
        ],
        out_specs=pl.BlockSpec((M, nt), lambda i: (0, i)),
        out_shape=jax.ShapeDtypeStruct((M, N), jnp.float32),
        scratch_shapes=[pltpu.VMEM((M, K), jnp.float32)],
    )(x, g.reshape(1, -1), be.reshape(1, -1), w, b.reshape(1, -1))


def _final_kernel(hp_ref, gp_ref, bep_ref, wp_ref, bp_ref,
                  hv_ref, gv_ref, bev_ref, wv_ref, bv_ref,
                  pol_ref, val_ref):
    ap = _ln_relu(hp_ref[...], gp_ref[...], bep_ref[...])
    pol_ref[...] = (jnp.dot(ap, wp_ref[...],
                            preferred_element_type=jnp.float32, precision=jax.lax.Precision.HIGHEST) + bp_ref[...])
    av = _ln_relu(hv_ref[...], gv_ref[...], bev_ref[...])
    val_ref[...] = jnp.tanh(
        jnp.dot(av, wv_ref[...], preferred_element_type=jnp.float32, precision=jax.lax.Precision.HIGHEST)
        + bv_ref[...])


def kernel(observations, params):
    Bsz = observations.shape[0]
    obsT = observations.reshape(Bsz, N_NODES, N_NODES).transpose(0, 2, 1)
    obsT = obsT.reshape(Bsz * N_NODES, N_NODES)

    feat = _run_convs(observations, obsT,
                      params["convs"]).reshape(Bsz, N_NODES * CH)

    pp, vp = params["policy"], params["value"]
    h1p = _gemm_bias(feat, pp["Ws"][0], pp["bs"][0], 1024)
    h1v = _gemm_bias(feat, vp["Ws"][0], vp["bs"][0], 1024)
    h2p = _ln_gemm_bias(h1p, pp["gammas"][0], pp["betas"][0],
                        pp["Ws"][1], pp["bs"][1], 512)
    h2v = _ln_gemm_bias(h1v, vp["gammas"][0], vp["betas"][0],
                        vp["Ws"][1], vp["bs"][1], 512)

    policy, value = pl.pallas_call(
        _final_kernel,
        out_shape=(
            jax.ShapeDtypeStruct((Bsz, pp["Ws"][2].shape[1]), jnp.float32),
            jax.ShapeDtypeStruct((Bsz, 1), jnp.float32),
        ),
    )(h2p, pp["gammas"][1].reshape(1, -1), pp["betas"][1].reshape(1, -1),
      pp["Ws"][2], pp["bs"][2].reshape(1, -1),
      h2v, vp["gammas"][1].reshape(1, -1), vp["betas"][1].reshape(1, -1),
      vp["Ws"][2], vp["bs"][2].reshape(1, -1))
    return (policy, value)


# speedup vs baseline: 2.3268x; 1.4281x over previous
import jax
import jax.numpy as jnp
from jax.experimental import pallas as pl
from jax.experimental.pallas import tpu as pltpu

N_NODES = 48
CH = 32
B = 64


def _bn(x, g, b):
    m = jnp.mean(x, axis=0, keepdims=True)
    v = jnp.mean((x - m) ** 2, axis=0, keepdims=True)
    return (x - m) / jnp.sqrt(v + 1e-5) * g + b


def _ln_relu(x, g, b):
    m = jnp.mean(x, axis=-1, keepdims=True)
    v = jnp.mean((x - m) ** 2, axis=-1, keepdims=True)
    return jax.nn.relu((x - m) / jnp.sqrt(v + 1e-5) * g + b)


def _dot_bf16(a, b):
    return jnp.dot(a.astype(jnp.bfloat16), b.astype(jnp.bfloat16),
                   preferred_element_type=jnp.float32)


def _conv_mlp(t, Ws, bs, gs, betas):
    for li in range(4):
        t = _dot_bf16(t, Ws[li]) + bs[li]
        if li < 3:
            t = jax.nn.relu(_bn(t, gs[li], betas[li]))
    return t


def _lift0_kernel(obs_ref, eW_ref, eb_ref, out_ref):
    out_ref[...] = jax.nn.relu(1.0 + obs_ref[...] * eW_ref[0, 0] + eb_ref[0, 0])


def _conv0_mlp_xla(h0, c):
    t = h0 @ c["Ws"][0] + c["bs"][0]
    for li in range(4):
        if li > 0:
            t = t @ c["Ws"][li] + c["bs"][li]
        if li < 3:
            m = jnp.mean(t, axis=0, keepdims=True)
            v = jnp.var(t, axis=0, keepdims=True)
            t = (t - m) / jnp.sqrt(v + 1e-5) * c["gammas"][li] + c["betas"][li]
            t = jax.nn.relu(t)
    return t


def _unpack(prefs):
    eW, eb = prefs[0][...], prefs[1][...]
    Ws = [r[...] for r in prefs[2:6]]
    bs = [r[...] for r in prefs[6:10]]
    gs = [r[...] for r in prefs[10:13]]
    betas = [r[...] for r in prefs[13:16]]
    return (eW, eb), Ws, bs, gs, betas


def _convN_kernel(nhalf, jk_in, obsT_ref, x_ref, x2d_ref, *refs):
    out_ref = refs[-1]
    jk_refs = refs[16:-1] if jk_in else ()
    (eW, eb), Ws, bs, gs, betas = _unpack(refs[:16])
    obsT = obsT_ref[...]
    x = x_ref[...]
    x2d = x2d_ref[...]
    M = B * N_NODES
    W = N_NODES * CH // nhalf

    aggr = jnp.zeros((M, CH), jnp.float32)
    for h in range(nhalf):
        qg = jax.lax.broadcasted_iota(jnp.int32, (N_NODES, W), 1) + h * W
        jr = jax.lax.broadcasted_iota(jnp.int32, (N_NODES, W), 0)
        csel = jax.lax.broadcasted_iota(jnp.int32, (CH, W), 1) + h * W
        crow = jax.lax.broadcasted_iota(jnp.int32, (CH, W), 0)
        Tsel = (csel % CH == crow).astype(jnp.float32)
        wtile = jnp.dot(eW, Tsel, preferred_element_type=jnp.float32, precision=jax.lax.Precision.HIGHEST)
        ebtile = jnp.dot(eb, Tsel, preferred_element_type=jnp.float32, precision=jax.lax.Precision.HIGHEST)
        R = jnp.where(jr == qg // CH, wtile, 0.0)
        O = jnp.dot(obsT, R, preferred_element_type=jnp.float32, precision=jax.lax.Precision.HIGHEST)
        xb = jnp.broadcast_to(
            x2d[:, None, h * W:(h + 1) * W], (B, N_NODES, W)).reshape(M, W)
        sq = jax.lax.broadcasted_iota(jnp.int32, (W, CH), 0) + h * W
        sc = jax.lax.broadcasted_iota(jnp.int32, (W, CH), 1)
        S = (sq % CH == sc).astype(jnp.float32)
        Z = jax.nn.relu(xb + ebtile + O)
        aggr = aggr + jnp.dot(Z, S, preferred_element_type=jnp.float32, precision=jax.lax.Precision.HIGHEST)
    hcat = x + aggr
    t = _conv_mlp(hcat, Ws, bs, gs, betas)
    for jr_ref in jk_refs:
        t = jnp.maximum(t, jr_ref[...])
    out_ref[...] = t


def _conv_args(c):
    args = [c["eW"], c["eb"].reshape(1, -1)]
    args.extend(c["Ws"])
    args.extend(b.reshape(1, -1) for b in c["bs"])
    args.extend(g.reshape(1, -1) for g in c["gammas"])
    args.extend(b.reshape(1, -1) for b in c["betas"])
    return args


def _run_convs(observations, obsT, convs):
    import functools
    M = B * N_NODES
    oshape = jax.ShapeDtypeStruct((M, CH), jnp.float32)
    c0 = convs[0]
    msgE = pl.pallas_call(
        _lift0_kernel,
        out_shape=jax.ShapeDtypeStruct(observations.shape, jnp.float32),
    )(observations, c0["eW"], c0["eb"].reshape(1, -1))
    dst = jnp.arange(observations.shape[1]) % N_NODES
    aggr0 = jnp.zeros((B, N_NODES), jnp.float32).at[:, dst].add(msgE)
    h0 = (1.0 + aggr0).reshape(M, 1)
    x1 = _conv0_mlp_xla(h0, convs[0])
    x2 = pl.pallas_call(functools.partial(_convN_kernel, 2, False),
                        out_shape=oshape)(
        obsT, x1, x1.reshape(B, N_NODES * CH), *_conv_args(convs[1]))
    jk = pl.pallas_call(functools.partial(_convN_kernel, 2, True),
                        out_shape=oshape)(
        obsT, x2, x2.reshape(B, N_NODES * CH), *_conv_args(convs[2]), x1, x2)
    return jk


def _gemm_bias_kernel(x_ref, w_ref, b_ref, o_ref):
    o_ref[...] = (jnp.dot(x_ref[...], w_ref[...],
                          preferred_element_type=jnp.float32) + b_ref[...])


def _gemm_bias(x, w, b, nt):
    K, N = w.shape
    return pl.pallas_call(
        _gemm_bias_kernel,
        grid=(N // nt,),
        in_specs=[
            pl.BlockSpec((x.shape[0], K), lambda i: (0, 0)),
            pl.BlockSpec((K, nt), lambda i: (0, i)),
            pl.BlockSpec((1, nt), lambda i: (0, i)),
        ],
        out_specs=pl.BlockSpec((x.shape[0], nt), lambda i: (0, i)),
        out_shape=jax.ShapeDtypeStruct((x.shape[0], N), jnp.float32),
    )(x, w, b.reshape(1, -1))


def _ln_gemm_kernel(x_ref, g_ref, be_ref, w_ref, b_ref, o_ref, a_scr):
    @pl.when(pl.program_id(0) == 0)
    def _():
        a_scr[...] = _ln_relu(x_ref[...], g_ref[...], be_ref[...])

    o_ref[...] = (jnp.dot(a_scr[...], w_ref[...],
                          preferred_element_type=jnp.float32) + b_ref[...])


def _ln_gemm_bias(x, g, be, w, b, nt):
    M, K = x.shape
    N = w.shape[1]
    return pl.pallas_call(
        _ln_gemm_kernel,
        grid=(N // nt,),
        in_specs=[
            pl.BlockSpec((M, K), lambda i: (0, 0)),
            pl.BlockSpec((1, K), lambda i: (0, 0)),
            pl.BlockSpec((1, K), lambda i: (0, 0)),
            pl.BlockSpec((K, nt), lambda i: (0, i)),
            pl.BlockSpec((1, nt), lambda i: (0, i)),
        ],
        out_specs=pl.BlockSpec((M, nt), lambda i: (0, i)),
        out_shape=jax.ShapeDtypeStruct((M, N), jnp.float32),
        scratch_shapes=[pltpu.VMEM((M, K), jnp.float32)],
    )(x, g.reshape(1, -1), be.reshape(1, -1), w, b.reshape(1, -1))


def _final_kernel(hp_ref, gp_ref, bep_ref, wp_ref, bp_ref,
                  hv_ref, gv_ref, bev_ref, wv_ref, bv_ref,
                  pol_ref, val_ref):
    ap = _ln_relu(hp_ref[...], gp_ref[...], bep_ref[...])
    pol_ref[...] = (jnp.dot(ap, wp_ref[...],
                            preferred_element_type=jnp.float32) + bp_ref[...])
    av = _ln_relu(hv_ref[...], gv_ref[...], bev_ref[...])
    val_ref[...] = jnp.tanh(
        jnp.dot(av, wv_ref[...], preferred_element_type=jnp.float32)
        + bv_ref[...])


def kernel(observations, params):
    Bsz = observations.shape[0]
    obsT = observations.reshape(Bsz, N_NODES, N_NODES).transpose(0, 2, 1)
    obsT = obsT.reshape(Bsz * N_NODES, N_NODES)

    feat = _run_convs(observations, obsT,
                      params["convs"]).reshape(Bsz, N_NODES * CH)

    pp, vp = params["policy"], params["value"]
    h1p = _gemm_bias(feat, pp["Ws"][0], pp["bs"][0], 1024)
    h1v = _gemm_bias(feat, vp["Ws"][0], vp["bs"][0], 1024)
    h2p = _ln_gemm_bias(h1p, pp["gammas"][0], pp["betas"][0],
                        pp["Ws"][1], pp["bs"][1], 512)
    h2v = _ln_gemm_bias(h1v, vp["gammas"][0], vp["betas"][0],
                        vp["Ws"][1], vp["bs"][1], 512)

    policy, value = pl.pallas_call(
        _final_kernel,
        out_shape=(
            jax.ShapeDtypeStruct((Bsz, pp["Ws"][2].shape[1]), jnp.float32),
            jax.ShapeDtypeStruct((Bsz, 1), jnp.float32),
        ),
    )(h2p, pp["gammas"][1].reshape(1, -1), pp["betas"][1].reshape(1, -1),
      pp["Ws"][2], pp["bs"][2].reshape(1, -1),
      h2v, vp["gammas"][1].reshape(1, -1), vp["betas"][1].reshape(1, -1),
      vp["Ws"][2], vp["bs"][2].reshape(1, -1))
    return (policy, value)


# speedup vs baseline: 2.3297x; 1.0012x over previous
import jax
import jax.numpy as jnp
from jax.experimental import pallas as pl
from jax.experimental.pallas import tpu as pltpu

N_NODES = 48
CH = 32
B = 64


def _bn(x, g, b):
    m = jnp.mean(x, axis=0, keepdims=True)
    v = jnp.mean((x - m) ** 2, axis=0, keepdims=True)
    return (x - m) / jnp.sqrt(v + 1e-5) * g + b


def _ln_relu(x, g, b):
    m = jnp.mean(x, axis=-1, keepdims=True)
    v = jnp.mean((x - m) ** 2, axis=-1, keepdims=True)
    return jax.nn.relu((x - m) / jnp.sqrt(v + 1e-5) * g + b)


def _dot_bf16(a, b):
    return jnp.dot(a.astype(jnp.bfloat16), b.astype(jnp.bfloat16),
                   preferred_element_type=jnp.float32)


def _conv_mlp(t, Ws, bs, gs, betas):
    for li in range(4):
        t = _dot_bf16(t, Ws[li]) + bs[li]
        if li < 3:
            t = jax.nn.relu(_bn(t, gs[li], betas[li]))
    return t


def _lift0_kernel(obs_ref, eW_ref, eb_ref, out_ref):
    out_ref[...] = jax.nn.relu(1.0 + obs_ref[...] * eW_ref[0, 0] + eb_ref[0, 0])


def _conv0_mlp_xla(h0, c):
    t = h0 @ c["Ws"][0] + c["bs"][0]
    for li in range(4):
        if li > 0:
            t = t @ c["Ws"][li] + c["bs"][li]
        if li < 3:
            m = jnp.mean(t, axis=0, keepdims=True)
            v = jnp.var(t, axis=0, keepdims=True)
            t = (t - m) / jnp.sqrt(v + 1e-5) * c["gammas"][li] + c["betas"][li]
            t = jax.nn.relu(t)
    return t


def _unpack(prefs):
    eW, eb = prefs[0][...], prefs[1][...]
    Ws = [r[...] for r in prefs[2:6]]
    bs = [r[...] for r in prefs[6:10]]
    gs = [r[...] for r in prefs[10:13]]
    betas = [r[...] for r in prefs[13:16]]
    return (eW, eb), Ws, bs, gs, betas


def _convN_kernel(nhalf, jk_in, obsT_ref, x_ref, x2d_ref, *refs):
    out_ref = refs[-1]
    jk_refs = refs[16:-1] if jk_in else ()
    (eW, eb), Ws, bs, gs, betas = _unpack(refs[:16])
    obsT = obsT_ref[...]
    x = x_ref[...]
    x2d = x2d_ref[...]
    M = B * N_NODES
    W = N_NODES * CH // nhalf

    csel = jax.lax.broadcasted_iota(jnp.int32, (CH, N_NODES * CH), 1)
    crow = jax.lax.broadcasted_iota(jnp.int32, (CH, N_NODES * CH), 0)
    Tfull = (csel % CH == crow).astype(jnp.float32)
    ebtile = jnp.dot(eb, Tfull, preferred_element_type=jnp.float32,
                     precision=jax.lax.Precision.HIGHEST)
    xp = x2d + ebtile
    aggr = jnp.zeros((M, CH), jnp.float32)
    for h in range(nhalf):
        qg = jax.lax.broadcasted_iota(jnp.int32, (N_NODES, W), 1) + h * W
        jr = jax.lax.broadcasted_iota(jnp.int32, (N_NODES, W), 0)
        wtile = jnp.dot(eW, Tfull[:, h * W:(h + 1) * W],
                        preferred_element_type=jnp.float32,
                        precision=jax.lax.Precision.HIGHEST)
        R = jnp.where(jr == qg // CH, wtile, 0.0)
        O = jnp.dot(obsT, R, preferred_element_type=jnp.float32, precision=jax.lax.Precision.HIGHEST)
        Z = jax.nn.relu(xp[:, None, h * W:(h + 1) * W]
                        + O.reshape(B, N_NODES, W)).reshape(M, W)
        sq = jax.lax.broadcasted_iota(jnp.int32, (W, CH), 0) + h * W
        sc = jax.lax.broadcasted_iota(jnp.int32, (W, CH), 1)
        S = (sq % CH == sc).astype(jnp.float32)
        aggr = aggr + jnp.dot(Z, S, preferred_element_type=jnp.float32, precision=jax.lax.Precision.HIGHEST)
    hcat = x + aggr
    t = _conv_mlp(hcat, Ws, bs, gs, betas)
    for jr_ref in jk_refs:
        t = jnp.maximum(t, jr_ref[...])
    out_ref[...] = t


def _conv_args(c):
    args = [c["eW"], c["eb"].reshape(1, -1)]
    args.extend(c["Ws"])
    args.extend(b.reshape(1, -1) for b in c["bs"])
    args.extend(g.reshape(1, -1) for g in c["gammas"])
    args.extend(b.reshape(1, -1) for b in c["betas"])
    return args


def _run_convs(observations, obsT, convs):
    import functools
    M = B * N_NODES
    oshape = jax.ShapeDtypeStruct((M, CH), jnp.float32)
    c0 = convs[0]
    msgE = pl.pallas_call(
        _lift0_kernel,
        out_shape=jax.ShapeDtypeStruct(observations.shape, jnp.float32),
    )(observations, c0["eW"], c0["eb"].reshape(1, -1))
    dst = jnp.arange(observations.shape[1]) % N_NODES
    aggr0 = jnp.zeros((B, N_NODES), jnp.float32).at[:, dst].add(msgE)
    h0 = (1.0 + aggr0).reshape(M, 1)
    x1 = _conv0_mlp_xla(h0, convs[0])
    x2 = pl.pallas_call(functools.partial(_convN_kernel, 2, False),
                        out_shape=oshape)(
        obsT, x1, x1.reshape(B, N_NODES * CH), *_conv_args(convs[1]))
    jk = pl.pallas_call(functools.partial(_convN_kernel, 2, True),
                        out_shape=oshape)(
        obsT, x2, x2.reshape(B, N_NODES * CH), *_conv_args(convs[2]), x1, x2)
    return jk


def _gemm_bias_kernel(x_ref, w_ref, b_ref, o_ref):
    o_ref[...] = (jnp.dot(x_ref[...], w_ref[...],
                          preferred_element_type=jnp.float32) + b_ref[...])


def _gemm_bias(x, w, b, nt):
    K, N = w.shape
    return pl.pallas_call(
        _gemm_bias_kernel,
        grid=(N // nt,),
        in_specs=[
            pl.BlockSpec((x.shape[0], K), lambda i: (0, 0)),
            pl.BlockSpec((K, nt), lambda i: (0, i)),
            pl.BlockSpec((1, nt), lambda i: (0, i)),
        ],
        out_specs=pl.BlockSpec((x.shape[0], nt), lambda i: (0, i)),
        out_shape=jax.ShapeDtypeStruct((x.shape[0], N), jnp.float32),
    )(x, w, b.reshape(1, -1))


def _ln_gemm_kernel(x_ref, g_ref, be_ref, w_ref, b_ref, o_ref, a_scr):
    @pl.when(pl.program_id(0) == 0)
    def _():
        a_scr[...] = _ln_relu(x_ref[...], g_ref[...], be_ref[...])

    o_ref[...] = (jnp.dot(a_scr[...], w_ref[...],
                          preferred_element_type=jnp.float32) + b_ref[...])


def _ln_gemm_bias(x, g, be, w, b, nt):
    M, K = x.shape
    N = w.shape[1]
    return pl.pallas_call(
        _ln_gemm_kernel,
        grid=(N // nt,),
        in_specs=[
            pl.BlockSpec((M, K), lambda i: (0, 0)),
            pl.BlockSpec((1, K), lambda i: (0, 0)),
            pl.BlockSpec((1, K), lambda i: (0, 0)),
            pl.BlockSpec((K, nt), lambda i: (0, i)),
            pl.BlockSpec((1, nt), lambda i: (0, i)),
        ],
        out_specs=pl.BlockSpec((M, nt), lambda i: (0, i)),
        out_shape=jax.ShapeDtypeStruct((M, N), jnp.float32),
        scratch_shapes=[pltpu.VMEM((M, K), jnp.float32)],
    )(x, g.reshape(1, -1), be.reshape(1, -1), w, b.reshape(1, -1))


def _final_kernel(hp_ref, gp_ref, bep_ref, wp_ref, bp_ref,
                  hv_ref, gv_ref, bev_ref, wv_ref, bv_ref,
                  pol_ref, val_ref):
    ap = _ln_relu(hp_ref[...], gp_ref[...], bep_ref[...])
    pol_ref[...] = (jnp.dot(ap, wp_ref[...],
                            preferred_element_type=jnp.float32) + bp_ref[...])
    av = _ln_relu(hv_ref[...], gv_ref[...], bev_ref[...])
    val_ref[...] = jnp.tanh(
        jnp.dot(av, wv_ref[...], preferred_element_type=jnp.float32)
        + bv_ref[...])


def kernel(observations, params):
    Bsz = observations.shape[0]
    obsT = observations.reshape(Bsz, N_NODES, N_NODES).transpose(0, 2, 1)
    obsT = obsT.reshape(Bsz * N_NODES, N_NODES)

    feat = _run_convs(observations, obsT,
                      params["convs"]).reshape(Bsz, N_NODES * CH)

    pp, vp = params["policy"], params["value"]
    h1p = _gemm_bias(feat, pp["Ws"][0], pp["bs"][0], 1024)
    h1v = _gemm_bias(feat, vp["Ws"][0], vp["bs"][0], 1024)
    h2p = _ln_gemm_bias(h1p, pp["gammas"][0], pp["betas"][0],
                        pp["Ws"][1], pp["bs"][1], 512)
    h2v = _ln_gemm_bias(h1v, vp["gammas"][0], vp["betas"][0],
                        vp["Ws"][1], vp["bs"][1], 512)

    policy, value = pl.pallas_call(
        _final_kernel,
        out_shape=(
            jax.ShapeDtypeStruct((Bsz, pp["Ws"][2].shape[1]), jnp.float32),
            jax.ShapeDtypeStruct((Bsz, 1), jnp.float32),
        ),
    )(h2p, pp["gammas"][1].reshape(1, -1), pp["betas"][1].reshape(1, -1),
      pp["Ws"][2], pp["bs"][2].reshape(1, -1),
      h2v, vp["gammas"][1].reshape(1, -1), vp["betas"][1].reshape(1, -1),
      vp["Ws"][2], vp["bs"][2].reshape(1, -1))
    return (policy, value)
